# baseline (speedup 1.0000x reference)
import functools

import jax
import jax.numpy as jnp
from jax import lax
from jax.experimental import pallas as pl
from jax.experimental.pallas import tpu as pltpu
from jax.experimental.pallas import tpu_sc as plsc

_N = 900
_NPAD = 1024
_NV = _NPAD // 16
_BIG = 1e30
_NB_SC = 4


def _tc_pairs_body(c_ref, ct_ref, out_ref):
    col = c_ref[0]
    ct = ct_ref[0]

    cx_c, cy_c = col[:, 0:1], col[:, 1:2]
    h_c, w_c = col[:, 2:3], col[:, 3:4]
    x1_c = cx_c - 0.5 * w_c
    y1_c = cy_c - 0.5 * h_c
    x2_c = cx_c + 0.5 * w_c
    y2_c = cy_c + 0.5 * h_c

    cx_r, cy_r = ct[0:1, :], ct[1:2, :]
    h_r, w_r = ct[2:3, :], ct[3:4, :]
    x1_r = cx_r - 0.5 * w_r
    y1_r = cy_r - 0.5 * h_r
    x2_r = cx_r + 0.5 * w_r
    y2_r = cy_r + 0.5 * h_r

    iw = jnp.maximum(jnp.maximum(x2_c, x2_r) - jnp.maximum(x1_c, x1_r), 0.0)
    ih = jnp.maximum(jnp.maximum(y2_c, y2_r) - jnp.maximum(y1_c, y1_r), 0.0)
    inter = iw * ih

    area_c = (x2_c - x1_c) * (y2_c - y1_c)
    area_r = (x2_r - x1_r) * (y2_r - y1_r)
    union = area_c + area_r - inter

    ii = lax.broadcasted_iota(jnp.int32, (_N, _N), 0)
    jj = lax.broadcasted_iota(jnp.int32, (_N, _N), 1)
    iou = inter / union - jnp.where(ii == jj, 1.0, 0.0)

    m = jnp.max(iou, axis=1, keepdims=True)
    am = jnp.min(jnp.where(iou == m, jj, _N), axis=1, keepdims=True)
    am = jnp.minimum(am, _N - 1)

    l1_c = jnp.abs(x2_c - x1_c) + jnp.abs(y2_c - y1_c)
    l1_r = jnp.abs(x2_r - x1_r) + jnp.abs(y2_r - y1_r)
    gathered = jnp.max(jnp.where(jj == am, jnp.broadcast_to(l1_r, (_N, _N)), 0.0),
                       axis=1, keepdims=True)

    idx0 = lax.broadcasted_iota(jnp.int32, (_N, 1), 0)
    keep = l1_c >= gathered
    out0 = jnp.where(keep, idx0, am)
    out1 = jnp.where(keep, am, idx0)
    out_ref[0] = jnp.concatenate([out0, out1], axis=1)


def _tc_pairs(top_k_centers, ct):
    b = top_k_centers.shape[0]
    return pl.pallas_call(
        _tc_pairs_body,
        grid=(b,),
        in_specs=[
            pl.BlockSpec((1, _N, 4), lambda i: (i, 0, 0)),
            pl.BlockSpec((1, 4, _N), lambda i: (i, 0, 0)),
        ],
        out_specs=pl.BlockSpec((1, _N, 2), lambda i: (i, 0, 0)),
        out_shape=jax.ShapeDtypeStruct((b, _N, 2), jnp.int32),
    )(top_k_centers, ct)


def _sc_body(wpb, rpw, ct_ref, out0_ref, out1_ref,
             ctv, x1v, y1v, x2v, y2v, av, l1v, o0v, o1v):
    wid = lax.axis_index("s") * 2 + lax.axis_index("c")
    b = wid // wpb
    r0 = (wid % wpb) * rpw
    pltpu.sync_copy(ct_ref.at[b], ctv)

    def stage(jv, _):
        sl = pl.ds(jv * 16, 16)
        cx = ctv[0, sl]
        cy = ctv[1, sl]
        h = ctv[2, sl]
        w = ctv[3, sl]
        x1 = cx - 0.5 * w
        y1 = cy - 0.5 * h
        x2 = cx + 0.5 * w
        y2 = cy + 0.5 * h
        x1v[sl] = x1
        y1v[sl] = y1
        x2v[sl] = x2
        y2v[sl] = y2
        av[sl] = (x2 - x1) * (y2 - y1)
        l1v[sl] = jnp.abs(x2 - x1) + jnp.abs(y2 - y1)
        return 0

    lax.fori_loop(0, _NV, stage, 0)
    lanes = lax.broadcasted_iota(jnp.int32, (16,), 0)

    def rowgrp(g, _):
        base = r0 + g * 16
        gsl = pl.ds(base, 16)
        gx1 = x1v[gsl]
        gy1 = y1v[gsl]
        gx2 = x2v[gsl]
        gy2 = y2v[gsl]
        ga = av[gsl]
        gl1 = l1v[gsl]
        acc0 = jnp.zeros((16,), jnp.int32)
        acc1 = jnp.zeros((16,), jnp.int32)
        for k in range(16):
            i = base + k
            sx1 = jnp.full((16,), gx1[k])
            sy1 = jnp.full((16,), gy1[k])
            sx2 = jnp.full((16,), gx2[k])
            sy2 = jnp.full((16,), gy2[k])
            sa = jnp.full((16,), ga[k])

            def inner(jv, carry):
                bn, bd, bi = carry
                sl = pl.ds(jv * 16, 16)
                iw = jnp.maximum(jnp.maximum(sx2, x2v[sl]) - jnp.maximum(sx1, x1v[sl]), 0.0)
                ih = jnp.maximum(jnp.maximum(sy2, y2v[sl]) - jnp.maximum(sy1, y1v[sl]), 0.0)
                inter = iw * ih
                d = (sa + av[sl]) - inter
                idxv = jv * 16 + lanes
                n = jnp.where(idxv == i, 0.0, inter)
                ok = d > 0.0
                n = jnp.where(ok, n, -1.0)
                d2 = jnp.where(ok, d, 1.0)
                gt = n * bd > bn * d2
                bn = jnp.where(gt, n, bn)
                bd = jnp.where(gt, d2, bd)
                bi = jnp.where(gt, idxv, bi)
                return bn, bd, bi

            bn, bd, bi = lax.fori_loop(
                0, _NV, inner,
                (jnp.full((16,), -1.0), jnp.full((16,), 1.0),
                 jnp.zeros((16,), jnp.int32)),
                unroll=4)
            q = bn / bd
            m = jnp.max(q)
            cand = jnp.where(q == m, bi, _NPAD)
            am = jnp.min(cand)
            l1am = plsc.load_gather(l1v, [jnp.full((16,), am)])
            keep = jnp.full((16,), gl1[k]) >= l1am
            iv = jnp.full((16,), i)
            amv = jnp.full((16,), am)
            v0 = jnp.where(keep, iv, amv)
            v1 = jnp.where(keep, amv, iv)
            sel = lanes == k
            acc0 = jnp.where(sel, v0, acc0)
            acc1 = jnp.where(sel, v1, acc1)
        o0v[pl.ds(g * 16, 16)] = acc0
        o1v[pl.ds(g * 16, 16)] = acc1
        return 0

    lax.fori_loop(0, rpw // 16, rowgrp, 0)
    pltpu.sync_copy(o0v, out0_ref.at[b, pl.ds(r0, rpw)])
    pltpu.sync_copy(o1v, out1_ref.at[b, pl.ds(r0, rpw)])


def _sc_pairs(ct_pad):
    nb = ct_pad.shape[0]
    wpb = 32 // nb
    rpw = _NPAD // wpb
    mesh = plsc.VectorSubcoreMesh(core_axis_name="c", subcore_axis_name="s")
    out0, out1 = pl.kernel(
        functools.partial(_sc_body, wpb, rpw),
        out_type=(jax.ShapeDtypeStruct((nb, _NPAD), jnp.int32),
                  jax.ShapeDtypeStruct((nb, _NPAD), jnp.int32)),
        mesh=mesh,
        scratch_types=[
            pltpu.VMEM((4, _NPAD), jnp.float32),
            pltpu.VMEM((_NPAD,), jnp.float32),
            pltpu.VMEM((_NPAD,), jnp.float32),
            pltpu.VMEM((_NPAD,), jnp.float32),
            pltpu.VMEM((_NPAD,), jnp.float32),
            pltpu.VMEM((_NPAD,), jnp.float32),
            pltpu.VMEM((_NPAD,), jnp.float32),
            pltpu.VMEM((rpw,), jnp.int32),
            pltpu.VMEM((rpw,), jnp.int32),
        ],
        compiler_params=pltpu.CompilerParams(needs_layout_passes=False),
    )(ct_pad)
    return jnp.stack([out0[:, :_N], out1[:, :_N]], axis=-1)


def kernel(query, key, value, top_k_centers):
    del query, key, value
    b = top_k_centers.shape[0]
    ct = jnp.transpose(top_k_centers, (0, 2, 1))
    n_tc = b - _NB_SC
    tc_out = _tc_pairs(top_k_centers[:n_tc], ct[:n_tc])
    pad = jnp.broadcast_to(
        jnp.array([_BIG, _BIG, 0.0, 0.0], jnp.float32)[None, :, None],
        (_NB_SC, 4, _NPAD - _N))
    ct_pad = jnp.concatenate([ct[n_tc:], pad], axis=2)
    sc_out = _sc_pairs(ct_pad)
    return jnp.concatenate([tc_out, sc_out], axis=0)


# speedup vs baseline: 1.0011x; 1.0011x over previous
import functools

import jax
import jax.numpy as jnp
from jax import lax
from jax.experimental import pallas as pl
from jax.experimental.pallas import tpu as pltpu
from jax.experimental.pallas import tpu_sc as plsc

_N = 900
_NPAD = 1024
_NV = _NPAD // 16
_BIG = 1e30
_NB_SC = 4


def _tc_pairs_body(c_ref, ct_ref, out_ref):
    col = c_ref[0]
    ct = ct_ref[0]

    cx_c, cy_c = col[:, 0:1], col[:, 1:2]
    h_c, w_c = col[:, 2:3], col[:, 3:4]
    x1_c = cx_c - 0.5 * w_c
    y1_c = cy_c - 0.5 * h_c
    x2_c = cx_c + 0.5 * w_c
    y2_c = cy_c + 0.5 * h_c

    cx_r, cy_r = ct[0:1, :], ct[1:2, :]
    h_r, w_r = ct[2:3, :], ct[3:4, :]
    x1_r = cx_r - 0.5 * w_r
    y1_r = cy_r - 0.5 * h_r
    x2_r = cx_r + 0.5 * w_r
    y2_r = cy_r + 0.5 * h_r

    iw = jnp.maximum(jnp.maximum(x2_c, x2_r) - jnp.maximum(x1_c, x1_r), 0.0)
    ih = jnp.maximum(jnp.maximum(y2_c, y2_r) - jnp.maximum(y1_c, y1_r), 0.0)
    inter = iw * ih

    area_c = (x2_c - x1_c) * (y2_c - y1_c)
    area_r = (x2_r - x1_r) * (y2_r - y1_r)
    union = area_c + area_r - inter

    ii = lax.broadcasted_iota(jnp.int32, (_N, _N), 0)
    jj = lax.broadcasted_iota(jnp.int32, (_N, _N), 1)
    iou = inter / union - jnp.where(ii == jj, 1.0, 0.0)

    m = jnp.max(iou, axis=1, keepdims=True)
    am = jnp.min(jnp.where(iou == m, jj, _N), axis=1, keepdims=True)
    am = jnp.minimum(am, _N - 1)

    l1_c = jnp.abs(x2_c - x1_c) + jnp.abs(y2_c - y1_c)
    l1_r = jnp.abs(x2_r - x1_r) + jnp.abs(y2_r - y1_r)
    gathered = jnp.max(jnp.where(jj == am, jnp.broadcast_to(l1_r, (_N, _N)), 0.0),
                       axis=1, keepdims=True)

    idx0 = lax.broadcasted_iota(jnp.int32, (_N, 1), 0)
    keep = l1_c >= gathered
    out0 = jnp.where(keep, idx0, am)
    out1 = jnp.where(keep, am, idx0)
    out_ref[0] = jnp.concatenate([out0, out1], axis=1)


def _tc_pairs(top_k_centers, ct):
    b = top_k_centers.shape[0]
    return pl.pallas_call(
        _tc_pairs_body,
        grid=(b,),
        in_specs=[
            pl.BlockSpec((1, _N, 4), lambda i: (i, 0, 0)),
            pl.BlockSpec((1, 4, _N), lambda i: (i, 0, 0)),
        ],
        out_specs=pl.BlockSpec((1, _N, 2), lambda i: (i, 0, 0)),
        out_shape=jax.ShapeDtypeStruct((b, _N, 2), jnp.int32),
    )(top_k_centers, ct)


def _sc_body(wpb, rpw, ct_ref, out0_ref, out1_ref,
             ctv, x1v, y1v, x2v, y2v, av, l1v, o0v, o1v):
    wid = lax.axis_index("s") * 2 + lax.axis_index("c")
    b = wid // wpb
    r0 = (wid % wpb) * rpw
    pltpu.sync_copy(ct_ref.at[b], ctv)

    def stage(jv, _):
        sl = pl.ds(jv * 16, 16)
        cx = ctv[0, sl]
        cy = ctv[1, sl]
        h = ctv[2, sl]
        w = ctv[3, sl]
        x1 = cx - 0.5 * w
        y1 = cy - 0.5 * h
        x2 = cx + 0.5 * w
        y2 = cy + 0.5 * h
        x1v[sl] = x1
        y1v[sl] = y1
        x2v[sl] = x2
        y2v[sl] = y2
        av[sl] = (x2 - x1) * (y2 - y1)
        l1v[sl] = jnp.abs(x2 - x1) + jnp.abs(y2 - y1)
        return 0

    lax.fori_loop(0, _NV, stage, 0)
    lanes = lax.broadcasted_iota(jnp.int32, (16,), 0)

    def rowgrp(g, _):
        base = r0 + g * 16
        gsl = pl.ds(base, 16)
        gx1 = x1v[gsl]
        gy1 = y1v[gsl]
        gx2 = x2v[gsl]
        gy2 = y2v[gsl]
        ga = av[gsl]
        gl1 = l1v[gsl]
        acc0 = jnp.zeros((16,), jnp.int32)
        acc1 = jnp.zeros((16,), jnp.int32)
        for k in range(16):
            i = base + k
            sx1 = jnp.full((16,), gx1[k])
            sy1 = jnp.full((16,), gy1[k])
            sx2 = jnp.full((16,), gx2[k])
            sy2 = jnp.full((16,), gy2[k])
            sa = jnp.full((16,), ga[k])

            def inner(jv, carry):
                bn, bd, bi = carry
                sl = pl.ds(jv * 16, 16)
                iw = jnp.maximum(jnp.maximum(sx2, x2v[sl]) - jnp.maximum(sx1, x1v[sl]), 0.0)
                ih = jnp.maximum(jnp.maximum(sy2, y2v[sl]) - jnp.maximum(sy1, y1v[sl]), 0.0)
                inter = iw * ih
                d = (sa + av[sl]) - inter
                idxv = jv * 16 + lanes
                n = jnp.where(idxv == i, 0.0, inter)
                ok = d > 0.0
                n = jnp.where(ok, n, -1.0)
                d2 = jnp.where(ok, d, 1.0)
                gt = n * bd > bn * d2
                bn = jnp.where(gt, n, bn)
                bd = jnp.where(gt, d2, bd)
                bi = jnp.where(gt, idxv, bi)
                return bn, bd, bi

            bn, bd, bi = lax.fori_loop(
                0, _NV, inner,
                (jnp.full((16,), -1.0), jnp.full((16,), 1.0),
                 jnp.zeros((16,), jnp.int32)),
                unroll=4)
            q = bn / bd
            m = jnp.max(q)
            cand = jnp.where(q == m, bi, _NPAD)
            am = jnp.min(cand)
            l1am = plsc.load_gather(l1v, [jnp.full((16,), am)])
            keep = jnp.full((16,), gl1[k]) >= l1am
            iv = jnp.full((16,), i)
            amv = jnp.full((16,), am)
            v0 = jnp.where(keep, iv, amv)
            v1 = jnp.where(keep, amv, iv)
            sel = lanes == k
            acc0 = jnp.where(sel, v0, acc0)
            acc1 = jnp.where(sel, v1, acc1)
        o0v[pl.ds(g * 16, 16)] = acc0
        o1v[pl.ds(g * 16, 16)] = acc1
        return 0

    lax.fori_loop(0, rpw // 16, rowgrp, 0)
    pltpu.sync_copy(o0v, out0_ref.at[b, pl.ds(r0, rpw)])
    pltpu.sync_copy(o1v, out1_ref.at[b, pl.ds(r0, rpw)])


def _sc_pairs(ct_pad):
    nb = ct_pad.shape[0]
    wpb = 32 // nb
    rpw = _NPAD // wpb
    mesh = plsc.VectorSubcoreMesh(core_axis_name="c", subcore_axis_name="s")
    out0, out1 = pl.kernel(
        functools.partial(_sc_body, wpb, rpw),
        out_type=(jax.ShapeDtypeStruct((nb, _NPAD), jnp.int32),
                  jax.ShapeDtypeStruct((nb, _NPAD), jnp.int32)),
        mesh=mesh,
        scratch_types=[
            pltpu.VMEM((4, _NPAD), jnp.float32),
            pltpu.VMEM((_NPAD,), jnp.float32),
            pltpu.VMEM((_NPAD,), jnp.float32),
            pltpu.VMEM((_NPAD,), jnp.float32),
            pltpu.VMEM((_NPAD,), jnp.float32),
            pltpu.VMEM((_NPAD,), jnp.float32),
            pltpu.VMEM((_NPAD,), jnp.float32),
            pltpu.VMEM((rpw,), jnp.int32),
            pltpu.VMEM((rpw,), jnp.int32),
        ],
        compiler_params=pltpu.CompilerParams(needs_layout_passes=False,
                                             skip_device_barrier=True),
    )(ct_pad)
    return jnp.stack([out0[:, :_N], out1[:, :_N]], axis=-1)


def kernel(query, key, value, top_k_centers):
    del query, key, value
    b = top_k_centers.shape[0]
    ct = jnp.transpose(top_k_centers, (0, 2, 1))
    n_tc = b - _NB_SC
    tc_out = _tc_pairs(top_k_centers[:n_tc], ct[:n_tc])
    pad = jnp.broadcast_to(
        jnp.array([_BIG, _BIG, 0.0, 0.0], jnp.float32)[None, :, None],
        (_NB_SC, 4, _NPAD - _N))
    ct_pad = jnp.concatenate([ct[n_tc:], pad], axis=2)
    sc_out = _sc_pairs(ct_pad)
    return jnp.concatenate([tc_out, sc_out], axis=0)
